# baseline (speedup 1.0000x reference)
import jax
import jax.numpy as jnp
from jax import lax
from jax.experimental import pallas as pl
from jax.experimental.pallas import tpu as pltpu
from jax.experimental.pallas import tpu_sc as plsc

_SZ = 96
_B, _D, _H, _W = 4, 128, 128, 128

_BZ = (28, 12, 5, 26)
_BY = (20, 17, 22, 23)
_BX = (4, 21, 4, 15)

_NC = 2
_NS = 16
_NW = _NC * _NS
_ZPW = _SZ // _NW

_NT_SC = 2
_CHUNK = _SZ * _W
_NBUF = 3


def _crop_body(vol, msk, o0, o1,
               b0, b1, b2, si0, si1, si2, so0, so1, so2):
    ins = (vol, msk)
    outs = (o0, o1)
    bufs = (b0, b1, b2)
    isems = (si0, si1, si2)
    osems = (so0, so1, so2)

    wid = lax.axis_index("s") * _NC + lax.axis_index("c")

    chunks = [(b, t, j)
              for b in range(_B) for t in range(_NT_SC) for j in range(_ZPW)]

    def in_start(b, j):
        z = wid * _ZPW + j
        return pl.multiple_of(((b * _D + _BZ[b] + z) * _H + _BY[b]) * _W, _W)

    def out_start(b, j):
        z = wid * _ZPW + j
        return pl.multiple_of((b * _SZ + z) * _CHUNK, _CHUNK)

    def issue_in(i):
        b, t, j = chunks[i]
        return pltpu.async_copy(
            ins[t].at[pl.ds(in_start(b, j), _CHUNK)], bufs[i % _NBUF],
            isems[i % _NBUF])

    def issue_out(i):
        b, t, j = chunks[i]
        return pltpu.async_copy(
            bufs[i % _NBUF], outs[t].at[pl.ds(out_start(b, j), _CHUNK)],
            osems[i % _NBUF])

    def repack(i):
        b, _, _ = chunks[i]
        bx = _BX[b]
        buf = bufs[i % _NBUF]

        def rows(y4, carry):
            for r in range(4):
                rbase = (y4 * 4 + r) * _W
                for k in range(6):
                    v = buf[pl.ds(rbase + bx + 16 * k, 16)]
                    buf[pl.ds(rbase + 16 * k, 16)] = v
            return carry

        lax.fori_loop(0, _SZ // 4, rows, 0)

    n = len(chunks)
    out_handles = [None] * n
    in_handles = [None] * n
    for i in range(_NBUF):
        in_handles[i] = issue_in(i)
    for i in range(n):
        in_handles[i].wait()
        repack(i)
        out_handles[i] = issue_out(i)
        if i + _NBUF < n:
            out_handles[i].wait()
            in_handles[i + _NBUF] = issue_in(i + _NBUF)
    for i in range(n - _NBUF, n):
        out_handles[i].wait()


def _tc_body(in0, in1, in2, in3, out):
    for k, ref in enumerate((in0, in1, in2, in3)):
        by, bx = _BY[k], _BX[k]
        out[k, 0, :, :_SZ] = ref[0, by:by + _SZ, bx:bx + _SZ]


@jax.jit
def _crop_call(vol, msk, skl):
    f32 = jnp.float32
    out_sds = jax.ShapeDtypeStruct((_B * _SZ * _SZ * _W,), f32)
    run = pl.kernel(
        _crop_body,
        out_type=[out_sds, out_sds],
        mesh=plsc.VectorSubcoreMesh(core_axis_name="c", subcore_axis_name="s"),
        scratch_types=[
            pltpu.VMEM((_CHUNK,), f32),
            pltpu.VMEM((_CHUNK,), f32),
            pltpu.VMEM((_CHUNK,), f32),
            pltpu.SemaphoreType.DMA,
            pltpu.SemaphoreType.DMA,
            pltpu.SemaphoreType.DMA,
            pltpu.SemaphoreType.DMA,
            pltpu.SemaphoreType.DMA,
            pltpu.SemaphoreType.DMA,
        ],
    )
    o0, o1 = run(vol.reshape(-1), msk.reshape(-1))

    in_specs = [
        pl.BlockSpec((1, _H, _W), lambda z, bz=_BZ[k]: (bz + z, 0, 0))
        for k in range(_B)
    ]
    o2 = pl.pallas_call(
        _tc_body,
        grid=(_SZ,),
        in_specs=in_specs,
        out_specs=pl.BlockSpec((_B, 1, _SZ, _W), lambda z: (0, z, 0, 0)),
        out_shape=jax.ShapeDtypeStruct((_B, _SZ, _SZ, _W), f32),
        compiler_params=pltpu.CompilerParams(
            dimension_semantics=("arbitrary",)),
    )(skl[0], skl[1], skl[2], skl[3])
    return o0, o1, o2


def kernel(volume, gt_mask, gt_skel):
    o0, o1, o2 = _crop_call(volume, gt_mask, gt_skel)
    shape = (_B, _SZ, _SZ, _SZ)

    def depad(o):
        return o.reshape(_B * _SZ * _SZ, _W)[:, :_SZ].reshape(shape)

    return (depad(o0), depad(o1), o2[..., :_SZ])


# speedup vs baseline: 1.8758x; 1.8758x over previous
import jax
import jax.numpy as jnp
from jax import lax
from jax.experimental import pallas as pl
from jax.experimental.pallas import tpu as pltpu
from jax.experimental.pallas import tpu_sc as plsc

_SZ = 96
_B, _D, _H, _W = 4, 128, 128, 128

_BZ = (28, 12, 5, 26)
_BY = (20, 17, 22, 23)
_BX = (4, 21, 4, 15)

_NC = 2
_NS = 16
_NW = _NC * _NS
_ZPW = _SZ // _NW

_CHUNK = _SZ * _W
_NBUF = 3


def _crop_body(vol, msk, skl, o0, o1, o2,
               b0, b1, b2, si0, si1, si2, so0, so1, so2):
    ins = (vol, msk, skl)
    outs = (o0, o1, o2)
    bufs = (b0, b1, b2)
    isems = (si0, si1, si2)
    osems = (so0, so1, so2)

    wid = lax.axis_index("s") * _NC + lax.axis_index("c")

    chunks = [(b, t, j) for b in range(_B) for t in range(3) for j in range(_ZPW)]

    def in_start(b, j):
        z = wid * _ZPW + j
        return pl.multiple_of(((b * _D + _BZ[b] + z) * _H + _BY[b]) * _W, _W)

    def out_start(b, j):
        z = wid * _ZPW + j
        return pl.multiple_of((b * _SZ + z) * _CHUNK, _CHUNK)

    def issue_in(i):
        b, t, j = chunks[i]
        return pltpu.async_copy(
            ins[t].at[pl.ds(in_start(b, j), _CHUNK)], bufs[i % _NBUF],
            isems[i % _NBUF])

    def issue_out(i):
        b, t, j = chunks[i]
        return pltpu.async_copy(
            bufs[i % _NBUF], outs[t].at[pl.ds(out_start(b, j), _CHUNK)],
            osems[i % _NBUF])

    def repack(i):
        b, _, _ = chunks[i]
        bx = _BX[b]
        buf = bufs[i % _NBUF]

        def rows(y4, carry):
            for r in range(4):
                rbase = (y4 * 4 + r) * _W
                for k in range(6):
                    v = buf[pl.ds(rbase + bx + 16 * k, 16)]
                    buf[pl.ds(rbase + 16 * k, 16)] = v
            return carry

        lax.fori_loop(0, _SZ // 4, rows, 0)

    n = len(chunks)
    out_handles = [None] * n
    in_handles = [None] * n
    for i in range(_NBUF):
        in_handles[i] = issue_in(i)
    for i in range(n):
        in_handles[i].wait()
        repack(i)
        out_handles[i] = issue_out(i)
        if i + _NBUF < n:
            out_handles[i].wait()
            in_handles[i + _NBUF] = issue_in(i + _NBUF)
    for i in range(n - _NBUF, n):
        out_handles[i].wait()


@jax.jit
def _crop_call(vol, msk, skl):
    f32 = jnp.float32
    out_sds = jax.ShapeDtypeStruct((_B * _SZ * _SZ * _W,), f32)
    run = pl.kernel(
        _crop_body,
        out_type=[out_sds, out_sds, out_sds],
        mesh=plsc.VectorSubcoreMesh(core_axis_name="c", subcore_axis_name="s"),
        scratch_types=[
            pltpu.VMEM((_CHUNK,), f32),
            pltpu.VMEM((_CHUNK,), f32),
            pltpu.VMEM((_CHUNK,), f32),
            pltpu.SemaphoreType.DMA,
            pltpu.SemaphoreType.DMA,
            pltpu.SemaphoreType.DMA,
            pltpu.SemaphoreType.DMA,
            pltpu.SemaphoreType.DMA,
            pltpu.SemaphoreType.DMA,
        ],
    )
    return run(vol.reshape(-1), msk.reshape(-1), skl.reshape(-1))


def kernel(volume, gt_mask, gt_skel):
    o0, o1, o2 = _crop_call(volume, gt_mask, gt_skel)
    shape = (_B, _SZ, _SZ, _SZ)

    def depad(o):
        return o.reshape(_B * _SZ * _SZ, _W)[:, :_SZ].reshape(shape)

    return (depad(o0), depad(o1), depad(o2))


# speedup vs baseline: 1.8818x; 1.0032x over previous
import jax
import jax.numpy as jnp
from jax import lax
from jax.experimental import pallas as pl
from jax.experimental.pallas import tpu as pltpu
from jax.experimental.pallas import tpu_sc as plsc

_SZ = 96
_B, _D, _H, _W = 4, 128, 128, 128

_BZ = (28, 12, 5, 26)
_BY = (20, 17, 22, 23)
_BX = (4, 21, 4, 15)

_NC = 2
_NS = 16
_NW = _NC * _NS
_ZPW = _SZ // _NW

_CHUNK = _SZ * _W
_NBUF = 4


def _crop_body(vol, msk, skl, o0, o1, o2,
               b0, b1, b2, b3, si0, si1, si2, si3, so0, so1, so2, so3):
    ins = (vol, msk, skl)
    outs = (o0, o1, o2)
    bufs = (b0, b1, b2, b3)
    isems = (si0, si1, si2, si3)
    osems = (so0, so1, so2, so3)

    wid = lax.axis_index("s") * _NC + lax.axis_index("c")

    chunks = [(b, t, j) for b in range(_B) for t in range(3) for j in range(_ZPW)]

    def in_start(b, j):
        z = wid * _ZPW + j
        return pl.multiple_of(((b * _D + _BZ[b] + z) * _H + _BY[b]) * _W, _W)

    def out_start(b, j):
        z = wid * _ZPW + j
        return pl.multiple_of((b * _SZ + z) * _CHUNK, _CHUNK)

    def issue_in(i):
        b, t, j = chunks[i]
        return pltpu.async_copy(
            ins[t].at[pl.ds(in_start(b, j), _CHUNK)], bufs[i % _NBUF],
            isems[i % _NBUF])

    def issue_out(i):
        b, t, j = chunks[i]
        return pltpu.async_copy(
            bufs[i % _NBUF], outs[t].at[pl.ds(out_start(b, j), _CHUNK)],
            osems[i % _NBUF])

    def repack(i):
        b, _, _ = chunks[i]
        bx = _BX[b]
        buf = bufs[i % _NBUF]

        def rows(y4, carry):
            for r in range(4):
                rbase = (y4 * 4 + r) * _W
                for k in range(6):
                    v = buf[pl.ds(rbase + bx + 16 * k, 16)]
                    buf[pl.ds(rbase + 16 * k, 16)] = v
            return carry

        lax.fori_loop(0, _SZ // 4, rows, 0)

    n = len(chunks)
    out_handles = [None] * n
    in_handles = [None] * n
    for i in range(_NBUF):
        in_handles[i] = issue_in(i)
    for i in range(n):
        in_handles[i].wait()
        repack(i)
        out_handles[i] = issue_out(i)
        if i + _NBUF < n:
            out_handles[i].wait()
            in_handles[i + _NBUF] = issue_in(i + _NBUF)
    for i in range(n - _NBUF, n):
        out_handles[i].wait()


@jax.jit
def _crop_call(vol, msk, skl):
    f32 = jnp.float32
    out_sds = jax.ShapeDtypeStruct((_B * _SZ * _SZ * _W,), f32)
    run = pl.kernel(
        _crop_body,
        out_type=[out_sds, out_sds, out_sds],
        mesh=plsc.VectorSubcoreMesh(core_axis_name="c", subcore_axis_name="s"),
        scratch_types=(
            [pltpu.VMEM((_CHUNK,), f32)] * _NBUF
            + [pltpu.SemaphoreType.DMA] * (2 * _NBUF)
        ),
    )
    return run(vol.reshape(-1), msk.reshape(-1), skl.reshape(-1))


def kernel(volume, gt_mask, gt_skel):
    o0, o1, o2 = _crop_call(volume, gt_mask, gt_skel)
    shape = (_B, _SZ, _SZ, _SZ)

    def depad(o):
        return o.reshape(_B * _SZ * _SZ, _W)[:, :_SZ].reshape(shape)

    return (depad(o0), depad(o1), depad(o2))
